# baseline (speedup 1.0000x reference)
import functools

import jax
import jax.numpy as jnp
from jax import lax
from jax.experimental import pallas as pl
from jax.experimental.pallas import tpu as pltpu
from jax.experimental.pallas import tpu_sc as plsc

E = 8
H = 1024
I = 2048
T = 512
G = 128
NB = T // G + E
NBUF = NB * G
NW = 32

_HI = lax.Precision.HIGHEST
_DN = (((1,), (0,)), ((), ()))
_DT = (((1,), (1,)), ((), ()))


def _fiota(shape, dim):
    return lax.broadcasted_iota(jnp.int32, shape, dim).astype(jnp.float32)

def _router_body(hs_ref, rw_ref, rs_ref, pos_ref, cm_ref, bm_ref, act_ref):
    hs = hs_ref[...]
    rw = rw_ref[...]
    lg = lax.dot_general(hs, rw, _DT, preferred_element_type=jnp.float32)
    mx = jnp.max(lg, axis=1, keepdims=True)
    e_row = _fiota((T, E), 1)
    idx = jnp.min(jnp.where(lg >= mx, e_row, float(E)), axis=1,
                  keepdims=True)
    oh = (e_row == idx).astype(jnp.float32)
    score = jax.nn.sigmoid(mx)

    id8 = (_fiota((E, E), 0) ==
           _fiota((E, E), 1)).astype(jnp.float32)
    rs_ref[...] = lax.dot_general(id8, oh * score, _DT, precision=_HI,
                                  preferred_element_type=jnp.float32)

    t_r = _fiota((T, T), 0)
    t_c = _fiota((T, T), 1)
    tri = (t_c <= t_r).astype(jnp.float32)
    rank = lax.dot_general(tri, oh, _DN, precision=_HI,
                           preferred_element_type=jnp.float32)
    counts = jnp.max(rank, axis=0, keepdims=True)
    pc = jnp.ceil(counts / G) * G
    u8 = (_fiota((E, E), 0) <=
          _fiota((E, E), 1)).astype(jnp.float32)
    incl = lax.dot_general(pc, u8, _DN, precision=_HI,
                           preferred_element_type=jnp.float32)
    off = incl - pc
    off_t = jnp.sum(oh * off, axis=1, keepdims=True)
    rank_t = jnp.sum(oh * rank, axis=1, keepdims=True)
    posf = off_t + rank_t - 1.0
    pos_ref[...] = posf.astype(jnp.int32)

    p_col = _fiota((T, NBUF), 1)
    cm_ref[...] = jnp.where(posf == p_col, score, 0.0)

    countsc = lax.dot_general(id8, counts, _DT, precision=_HI,
                              preferred_element_type=jnp.float32)
    pcc = jnp.ceil(countsc / G) * G
    l8 = (_fiota((E, E), 1) <=
          _fiota((E, E), 0)).astype(jnp.float32)
    inclc = lax.dot_general(l8, pcc, _DN, precision=_HI,
                            preferred_element_type=jnp.float32)
    b_col = _fiota((E, NB), 1) * G
    raw = jnp.sum((inclc <= b_col).astype(jnp.float32), axis=0,
                  keepdims=True)
    e_col1 = _fiota((E, 1), 0)
    last_e = jnp.max(jnp.where(countsc > 0.0, e_col1, -1.0))
    bm_ref[...] = jnp.minimum(raw, last_e).astype(jnp.int32)
    total = jnp.max(inclc)
    act_ref[...] = (b_col[0:1, :] < total).astype(jnp.int32)


def _moe_body(bm_ref, act_ref, cm_ref, hs_ref, g_ref, u_ref, dwa_ref, dwb_ref,
              y_ref):
    b = pl.program_id(0)

    @pl.when(act_ref[b] == 1)
    def _():
        cb = cm_ref[...].astype(jnp.bfloat16)
        x = lax.dot_general(cb, hs_ref[...].astype(jnp.bfloat16),
                            (((0,), (0,)), ((), ())),
                            preferred_element_type=jnp.float32
                            ).astype(jnp.bfloat16)
        g = jnp.dot(x, g_ref[0].astype(jnp.bfloat16),
                    preferred_element_type=jnp.float32)
        u = jnp.dot(x, u_ref[0].astype(jnp.bfloat16),
                    preferred_element_type=jnp.float32)
        a = (u * (g * jax.nn.sigmoid(g))).astype(jnp.bfloat16)
        y_ref[...] = (
            jnp.dot(a[:, :I // 2], dwa_ref[0].astype(jnp.bfloat16),
                    preferred_element_type=jnp.float32) +
            jnp.dot(a[:, I // 2:], dwb_ref[0].astype(jnp.bfloat16),
                    preferred_element_type=jnp.float32))


def _shared_body(hs_ref, gw_ref, uw_ref, dw_ref, o_ref):
    c = pl.program_id(0)
    hs = hs_ref[...].astype(jnp.bfloat16)
    g = lax.dot_general(hs, gw_ref[...].astype(jnp.bfloat16), _DT,
                        preferred_element_type=jnp.float32)
    u = lax.dot_general(hs, uw_ref[...].astype(jnp.bfloat16), _DT,
                        preferred_element_type=jnp.float32)
    a = u * (g * jax.nn.sigmoid(g))
    part = lax.dot_general(a.astype(jnp.bfloat16),
                           dw_ref[...].astype(jnp.bfloat16), _DT,
                           preferred_element_type=jnp.float32)

    @pl.when(c == 0)
    def _():
        o_ref[...] = part

    @pl.when(c != 0)
    def _():
        o_ref[...] += part


def _add_body(a_ref, b_ref, o_ref):
    o_ref[...] = a_ref[...] + b_ref[...]


def _make_sc_row_gather(n_out, rpw):
    mesh = plsc.VectorSubcoreMesh(core_axis_name="c", subcore_axis_name="s")

    @functools.partial(
        pl.kernel, mesh=mesh,
        out_type=jax.ShapeDtypeStruct((n_out, H), jnp.float32),
        scratch_types=[pltpu.VMEM((rpw,), jnp.int32),
                       pltpu.VMEM((rpw, H), jnp.float32),
                       pltpu.SemaphoreType.DMA])
    def k(src_hbm, idx_hbm, out_hbm, idx_v, rows_v, sem):
        wid = lax.axis_index("s") * 2 + lax.axis_index("c")
        base = wid * rpw
        pltpu.sync_copy(idx_hbm.at[pl.ds(base, rpw)], idx_v)
        pltpu.async_copy(src_hbm.at[idx_v], rows_v, sem).wait()
        pltpu.sync_copy(rows_v, out_hbm.at[pl.ds(base, rpw)])

    return k


def kernel(hidden_states, router_w, gate_up_proj, down_proj, gate_w, up_w,
           down_w):
    hs = hidden_states.reshape(-1, hidden_states.shape[-1])

    IC = I // 4
    shared = pl.pallas_call(
        _shared_body,
        grid=(I // IC,),
        in_specs=[
            pl.BlockSpec((T, H), lambda c: (0, 0)),
            pl.BlockSpec((IC, H), lambda c: (c, 0)),
            pl.BlockSpec((IC, H), lambda c: (c, 0)),
            pl.BlockSpec((H, IC), lambda c: (0, c)),
        ],
        out_specs=pl.BlockSpec((T, H), lambda c: (0, 0)),
        out_shape=jax.ShapeDtypeStruct((T, H), jnp.float32),
    )(hs, gate_w, up_w, down_w)

    hs, shared = lax.optimization_barrier((hs, shared))

    rs, pos, cmat, bm, act = pl.pallas_call(
        _router_body,
        out_shape=[
            jax.ShapeDtypeStruct((E, T), jnp.float32),
            jax.ShapeDtypeStruct((T, 1), jnp.int32),
            jax.ShapeDtypeStruct((T, NBUF), jnp.float32),
            jax.ShapeDtypeStruct((1, NB), jnp.int32),
            jax.ShapeDtypeStruct((1, NB), jnp.int32),
        ],
    )(hs, router_w)

    grid_spec = pltpu.PrefetchScalarGridSpec(
        num_scalar_prefetch=2,
        grid=(NB,),
        in_specs=[
            pl.BlockSpec((T, G), lambda b, bm, act: (0, b)),
            pl.BlockSpec((T, H), lambda b, bm, act: (0, 0)),
            pl.BlockSpec((1, H, I), lambda b, bm, act: (bm[b], 0, 0)),
            pl.BlockSpec((1, H, I), lambda b, bm, act: (bm[b], 0, 1)),
            pl.BlockSpec((1, I // 2, H), lambda b, bm, act: (bm[b], 0, 0)),
            pl.BlockSpec((1, I // 2, H), lambda b, bm, act: (bm[b], 1, 0)),
        ],
        out_specs=pl.BlockSpec((G, H), lambda b, bm, act: (b, 0)),
    )
    ybuf = pl.pallas_call(
        _moe_body,
        grid_spec=grid_spec,
        out_shape=jax.ShapeDtypeStruct((NBUF, H), jnp.float32),
    )(bm.reshape(NB), act.reshape(NB), cmat, hs, gate_up_proj, gate_up_proj,
      down_proj, down_proj)

    yun = _make_sc_row_gather(T, T // NW)(ybuf, pos.reshape(T))

    out = pl.pallas_call(
        _add_body,
        out_shape=jax.ShapeDtypeStruct((T, H), jnp.float32),
    )(shared, yun)

    return out, rs


# speedup vs baseline: 1.0242x; 1.0242x over previous
import functools

import jax
import jax.numpy as jnp
from jax import lax
from jax.experimental import pallas as pl
from jax.experimental.pallas import tpu as pltpu
from jax.experimental.pallas import tpu_sc as plsc

E = 8
H = 1024
I = 2048
T = 512
G = 128
NB = T // G + E
NBUF = NB * G
NW = 32

_HI = lax.Precision.HIGHEST
_DN = (((1,), (0,)), ((), ()))
_DT = (((1,), (1,)), ((), ()))


def _fiota(shape, dim):
    return lax.broadcasted_iota(jnp.int32, shape, dim).astype(jnp.float32)

def _router_body(hs_ref, rw_ref, rs_ref, pos_ref, cm_ref, bm_ref, act_ref):
    hs = hs_ref[...]
    rw = rw_ref[...]
    lg = lax.dot_general(hs, rw, _DT, preferred_element_type=jnp.float32)
    mx = jnp.max(lg, axis=1, keepdims=True)
    e_row = _fiota((T, E), 1)
    idx = jnp.min(jnp.where(lg >= mx, e_row, float(E)), axis=1,
                  keepdims=True)
    oh = (e_row == idx).astype(jnp.float32)
    score = jax.nn.sigmoid(mx)

    id8 = (_fiota((E, E), 0) ==
           _fiota((E, E), 1)).astype(jnp.float32)
    rs_ref[...] = lax.dot_general(id8, oh * score, _DT, precision=_HI,
                                  preferred_element_type=jnp.float32)

    t_r = _fiota((T, T), 0)
    t_c = _fiota((T, T), 1)
    tri = (t_c <= t_r).astype(jnp.float32)
    rank = lax.dot_general(tri, oh, _DN, precision=_HI,
                           preferred_element_type=jnp.float32)
    counts = jnp.max(rank, axis=0, keepdims=True)
    pc = jnp.ceil(counts / G) * G
    u8 = (_fiota((E, E), 0) <=
          _fiota((E, E), 1)).astype(jnp.float32)
    incl = lax.dot_general(pc, u8, _DN, precision=_HI,
                           preferred_element_type=jnp.float32)
    off = incl - pc
    off_t = jnp.sum(oh * off, axis=1, keepdims=True)
    rank_t = jnp.sum(oh * rank, axis=1, keepdims=True)
    posf = off_t + rank_t - 1.0
    pos_ref[...] = posf.astype(jnp.int32)

    p_col = _fiota((T, NBUF), 1)
    cm_ref[...] = jnp.where(posf == p_col, score, 0.0)

    countsc = lax.dot_general(id8, counts, _DT, precision=_HI,
                              preferred_element_type=jnp.float32)
    pcc = jnp.ceil(countsc / G) * G
    l8 = (_fiota((E, E), 1) <=
          _fiota((E, E), 0)).astype(jnp.float32)
    inclc = lax.dot_general(l8, pcc, _DN, precision=_HI,
                            preferred_element_type=jnp.float32)
    b_col = _fiota((E, NB), 1) * G
    raw = jnp.sum((inclc <= b_col).astype(jnp.float32), axis=0,
                  keepdims=True)
    e_col1 = _fiota((E, 1), 0)
    last_e = jnp.max(jnp.where(countsc > 0.0, e_col1, -1.0))
    bm_ref[...] = jnp.minimum(raw, last_e).astype(jnp.int32)
    total = jnp.max(inclc)
    act_ref[...] = (b_col[0:1, :] < total).astype(jnp.int32)


def _moe_body(bm_ref, act_ref, cm_ref, hs_ref, g_ref, u_ref, dwa_ref, dwb_ref,
              y_ref):
    b = pl.program_id(0)

    @pl.when(act_ref[b] == 1)
    def _():
        cb = cm_ref[...].astype(jnp.bfloat16)
        x = lax.dot_general(cb, hs_ref[...].astype(jnp.bfloat16),
                            (((0,), (0,)), ((), ())),
                            preferred_element_type=jnp.float32
                            ).astype(jnp.bfloat16)
        g = jnp.dot(x, g_ref[0].astype(jnp.bfloat16),
                    preferred_element_type=jnp.float32)
        u = jnp.dot(x, u_ref[0].astype(jnp.bfloat16),
                    preferred_element_type=jnp.float32)
        a = (u * (g * jax.nn.sigmoid(g))).astype(jnp.bfloat16)
        y_ref[...] = (
            jnp.dot(a[:, :I // 2], dwa_ref[0].astype(jnp.bfloat16),
                    preferred_element_type=jnp.float32) +
            jnp.dot(a[:, I // 2:], dwb_ref[0].astype(jnp.bfloat16),
                    preferred_element_type=jnp.float32))


def _shared_body(hs_ref, gw_ref, uw_ref, dw_ref, o_ref):
    c = pl.program_id(0)
    hs = hs_ref[...].astype(jnp.bfloat16)
    g = lax.dot_general(hs, gw_ref[...].astype(jnp.bfloat16), _DT,
                        preferred_element_type=jnp.float32)
    u = lax.dot_general(hs, uw_ref[...].astype(jnp.bfloat16), _DT,
                        preferred_element_type=jnp.float32)
    a = u * (g * jax.nn.sigmoid(g))
    part = lax.dot_general(a.astype(jnp.bfloat16),
                           dw_ref[...].astype(jnp.bfloat16), _DT,
                           preferred_element_type=jnp.float32)

    @pl.when(c == 0)
    def _():
        o_ref[...] = part

    @pl.when(c != 0)
    def _():
        o_ref[...] += part


def _add_body(a_ref, b_ref, o_ref):
    o_ref[...] = a_ref[...] + b_ref[...]


def _make_sc_row_gather(n_out, rpw):
    mesh = plsc.VectorSubcoreMesh(core_axis_name="c", subcore_axis_name="s")

    @functools.partial(
        pl.kernel, mesh=mesh,
        out_type=jax.ShapeDtypeStruct((n_out, H), jnp.float32),
        scratch_types=[pltpu.VMEM((rpw,), jnp.int32),
                       pltpu.VMEM((rpw, H), jnp.float32),
                       pltpu.SemaphoreType.DMA])
    def k(src_hbm, idx_hbm, out_hbm, idx_v, rows_v, sem):
        wid = lax.axis_index("s") * 2 + lax.axis_index("c")
        base = wid * rpw
        pltpu.sync_copy(idx_hbm.at[pl.ds(base, rpw)], idx_v)
        pltpu.async_copy(src_hbm.at[idx_v], rows_v, sem).wait()
        pltpu.sync_copy(rows_v, out_hbm.at[pl.ds(base, rpw)])

    return k


def kernel(hidden_states, router_w, gate_up_proj, down_proj, gate_w, up_w,
           down_w):
    hs = hidden_states.reshape(-1, hidden_states.shape[-1])

    rs, pos, cmat, bm, act = pl.pallas_call(
        _router_body,
        out_shape=[
            jax.ShapeDtypeStruct((E, T), jnp.float32),
            jax.ShapeDtypeStruct((T, 1), jnp.int32),
            jax.ShapeDtypeStruct((T, NBUF), jnp.float32),
            jax.ShapeDtypeStruct((1, NB), jnp.int32),
            jax.ShapeDtypeStruct((1, NB), jnp.int32),
        ],
    )(hs, router_w)

    grid_spec = pltpu.PrefetchScalarGridSpec(
        num_scalar_prefetch=2,
        grid=(NB,),
        in_specs=[
            pl.BlockSpec((T, G), lambda b, bm, act: (0, b)),
            pl.BlockSpec((T, H), lambda b, bm, act: (0, 0)),
            pl.BlockSpec((1, H, I), lambda b, bm, act: (bm[b], 0, 0)),
            pl.BlockSpec((1, H, I), lambda b, bm, act: (bm[b], 0, 1)),
            pl.BlockSpec((1, I // 2, H), lambda b, bm, act: (bm[b], 0, 0)),
            pl.BlockSpec((1, I // 2, H), lambda b, bm, act: (bm[b], 1, 0)),
        ],
        out_specs=pl.BlockSpec((G, H), lambda b, bm, act: (b, 0)),
    )
    ybuf = pl.pallas_call(
        _moe_body,
        grid_spec=grid_spec,
        out_shape=jax.ShapeDtypeStruct((NBUF, H), jnp.float32),
    )(bm.reshape(NB), act.reshape(NB), cmat, hs, gate_up_proj, gate_up_proj,
      down_proj, down_proj)

    yun = _make_sc_row_gather(T, T // NW)(ybuf, pos.reshape(T))

    IC = I // 4
    shared = pl.pallas_call(
        _shared_body,
        grid=(I // IC,),
        in_specs=[
            pl.BlockSpec((T, H), lambda c: (0, 0)),
            pl.BlockSpec((IC, H), lambda c: (c, 0)),
            pl.BlockSpec((IC, H), lambda c: (c, 0)),
            pl.BlockSpec((H, IC), lambda c: (0, c)),
        ],
        out_specs=pl.BlockSpec((T, H), lambda c: (0, 0)),
        out_shape=jax.ShapeDtypeStruct((T, H), jnp.float32),
    )(hs, gate_w, up_w, down_w)

    out = pl.pallas_call(
        _add_body,
        out_shape=jax.ShapeDtypeStruct((T, H), jnp.float32),
    )(shared, yun)

    return out, rs
